# baseline (speedup 1.0000x reference)
import functools

import jax
import jax.numpy as jnp
from jax import lax
from jax.experimental import pallas as pl
from jax.experimental.pallas import tpu as pltpu
from jax.experimental.pallas import tpu_sc as plsc


_ROW = 128
_N_OUT = 3


def _gather_rows(x_hbm, o0, o1, o2, sem):
    cid = lax.axis_index("c")
    sid = lax.axis_index("s")

    @pl.when(jnp.logical_and(cid == 0, sid == 0))
    def _():
        outs = (o0, o1, o2)
        copies = [
            pltpu.make_async_copy(x_hbm.at[i], outs[i], sem)
            for i in range(_N_OUT)
        ]
        for c in copies:
            c.start()
        for c in copies:
            c.wait()


def kernel(x):
    mesh = plsc.VectorSubcoreMesh(core_axis_name="c", subcore_axis_name="s")
    row = jax.ShapeDtypeStruct((_ROW,), jnp.float32)
    k = functools.partial(
        pl.kernel,
        mesh=mesh,
        out_type=(row, row, row),
        scratch_types=[pltpu.SemaphoreType.DMA],
    )(_gather_rows)
    return k(x)


# speedup vs baseline: 1.1223x; 1.1223x over previous
import functools

import jax
import jax.numpy as jnp
from jax import lax
from jax.experimental import pallas as pl
from jax.experimental.pallas import tpu as pltpu
from jax.experimental.pallas import tpu_sc as plsc


_ROW = 128
_N_OUT = 3


def _gather_rows(x_hbm, out_hbm):
    @pl.when(lax.axis_index("c") == 0)
    def _():
        pltpu.sync_copy(x_hbm.at[pl.ds(0, _N_OUT)], out_hbm)


def kernel(x):
    mesh = plsc.ScalarSubcoreMesh(axis_name="c", num_cores=1)
    k = functools.partial(
        pl.kernel,
        mesh=mesh,
        out_type=jax.ShapeDtypeStruct((_N_OUT, _ROW), jnp.float32),
    )(_gather_rows)
    out = k(x)
    return (out[0], out[1], out[2])
